# baseline (speedup 1.0000x reference)
import functools

import jax
import jax.numpy as jnp
from jax import lax
from jax.experimental import pallas as pl
from jax.experimental.pallas import tpu as pltpu
from jax.experimental.pallas import tpu_sc as plsc

BS = 16
L = 4096
D = 1024
NCORES = 2
NSUB = 16
CH = 96
DH = D // NCORES
NDB = DH // 16


def _avg_sc(seq, args, inv32):
    mesh = plsc.VectorSubcoreMesh(core_axis_name="c", subcore_axis_name="s")

    @functools.partial(
        pl.kernel,
        mesh=mesh,
        out_type=jax.ShapeDtypeStruct((BS, D), jnp.float32),
        scratch_types=[
            pltpu.VMEM((8, 2 * BS), jnp.int32),
            pltpu.VMEM((2 * BS,), jnp.float32),
            pltpu.VMEM((CH, DH), jnp.float32),
            pltpu.VMEM((CH, DH), jnp.float32),
            pltpu.VMEM((BS, DH), jnp.float32),
            pltpu.VMEM((NSUB, DH), jnp.float32),
            pltpu.VMEM_SHARED((NSUB, BS, DH), jnp.float32),
            pltpu.SemaphoreType.DMA,
            pltpu.SemaphoreType.DMA,
        ],
    )
    def k(seq_hbm, packed_hbm, inv_hbm, out_hbm,
          packed, inv_v, buf0, buf1, part, fin, shared, sem0, sem1):
        c = lax.axis_index("c")
        s = lax.axis_index("s")
        d0 = c * DH

        pltpu.sync_copy(packed_hbm, packed)
        pltpu.sync_copy(inv_hbm, inv_v)

        def ext(row, i):
            return packed[row, pl.ds(i, 16)][0]

        bg_v, en_v, cum_v, pw_v = 0, 1, 3, 4
        g0 = ext(pw_v, s)
        g1 = ext(pw_v, s + 1)

        def zero_part(r, carry):
            for db in range(NDB):
                part[r, pl.ds(db * 16, 16)] = jnp.zeros((16,), jnp.float32)
            return carry

        lax.fori_loop(0, BS, zero_part, 0)
        pltpu.sync_copy(part, shared.at[s])

        def start_dma(r, cb, buf, sem):
            pltpu.async_copy(
                seq_hbm.at[r, pl.ds(cb, CH), pl.ds(d0, DH)], buf, sem)

        def wait_dma(buf, sem):
            pltpu.make_async_copy(
                seq_hbm.at[0, pl.ds(0, CH), pl.ds(d0, DH)], buf, sem).wait()

        def chunk_base(g, base0):
            return jnp.minimum(base0 + g * CH, L - CH)

        def chunk(r, g, nch, base0, lo_abs, hi_abs, buf, sem):
            wait_dma(buf, sem)
            base = chunk_base(g, base0)
            lo = jnp.maximum(base0 + g * CH, lo_abs) - base
            hi = jnp.minimum(base0 + (g + 1) * CH, hi_abs) - base

            accs = tuple(part[r, pl.ds(db * 16, 16)] for db in range(NDB))

            def add_l(l, accs):
                return tuple(
                    a + buf[l, pl.ds(db * 16, 16)]
                    for db, a in enumerate(accs))

            n2 = (hi - lo) // 2

            def pair_body(i, accs):
                l = lo + 2 * i
                return add_l(l + 1, add_l(l, accs))

            accs = lax.fori_loop(0, n2, pair_body, accs)
            accs = lax.fori_loop(lo + 2 * n2, hi, add_l, accs)

            for db, a in enumerate(accs):
                part[r, pl.ds(db * 16, 16)] = a

            @pl.when(g + 2 < nch)
            def _():
                start_dma(r, chunk_base(g + 2, base0), buf, sem)

        def seg_bounds(r):
            S = ext(cum_v, r)
            bg_r = ext(bg_v, r)
            ln = ext(en_v, r) - bg_r
            a = jnp.maximum(g0 - S, 0)
            b = jnp.minimum(g1 - S, ln)
            return bg_r, a, b

        def seg_body(r, carry):
            bg_r, a, b = seg_bounds(r)

            @pl.when(a < b)
            def _():
                lo_abs = bg_r + a
                hi_abs = bg_r + b
                base0 = (lo_abs // 8) * 8
                nch = (hi_abs - base0 + CH - 1) // CH
                start_dma(r, chunk_base(0, base0), buf0, sem0)

                @pl.when(nch > 1)
                def _():
                    start_dma(r, chunk_base(1, base0), buf1, sem1)

                def g_body(g, carry2):
                    @pl.when(g % 2 == 0)
                    def _():
                        chunk(r, g, nch, base0, lo_abs, hi_abs, buf0, sem0)

                    @pl.when(g % 2 == 1)
                    def _():
                        chunk(r, g, nch, base0, lo_abs, hi_abs, buf1, sem1)

                    return carry2

                lax.fori_loop(0, nch, g_body, 0)

            return carry

        lax.fori_loop(0, BS, seg_body, 0)

        def copy_body(r, carry):
            _, a, b = seg_bounds(r)

            @pl.when(a < b)
            def _():
                pltpu.sync_copy(part.at[r], shared.at[s, r])

            return carry

        lax.fori_loop(0, BS, copy_body, 0)
        plsc.subcore_barrier()

        pltpu.sync_copy(shared.at[:, s, :], fin)
        inv = inv_v[pl.ds(s, 16)][0]
        for db in range(NDB):
            sl = pl.ds(db * 16, 16)
            a = fin[0, sl]
            for w in range(1, NSUB):
                a = a + fin[w, sl]
            fin[0, sl] = a * inv
        pltpu.sync_copy(fin.at[0], out_hbm.at[s, pl.ds(d0, DH)])

    return k(seq, args, inv32)


BLK = 256
NTB = L // BLK


NTC = 40


def _tc_blocks(seq, rows, blks, valid, inv_cnt):
    def body(rows_ref, blks_ref, valid_ref, inv_ref, seq_ref, out_ref):
        k = pl.program_id(0)

        @pl.when(k == 0)
        def _():
            out_ref[...] = jnp.zeros((BS, D), jnp.float32)

        @pl.when(valid_ref[k] == 1)
        def _():
            r = rows_ref[k]
            out_ref[pl.ds(r, 1), :] += jnp.sum(seq_ref[0], axis=0)[None, :]

        @pl.when(k == NTC - 1)
        def _():
            for r in range(BS):
                out_ref[pl.ds(r, 1), :] = (
                    out_ref[pl.ds(r, 1), :] * inv_ref[r])

    grid_spec = pltpu.PrefetchScalarGridSpec(
        num_scalar_prefetch=4,
        grid=(NTC,),
        in_specs=[pl.BlockSpec(
            (1, BLK, D),
            lambda k, rows, blks, valid, inv: (rows[k], blks[k], 0))],
        out_specs=pl.BlockSpec(
            (BS, D), lambda k, rows, blks, valid, inv: (0, 0)),
    )
    return pl.pallas_call(
        body, grid_spec=grid_spec,
        out_shape=jax.ShapeDtypeStruct((BS, D), jnp.float32),
    )(rows, blks, valid, inv_cnt, seq)


def kernel(seq, begin, end):
    begin = jnp.asarray(begin, jnp.int32)
    end = jnp.asarray(end, jnp.int32)
    lens = end - begin
    inv_cnt = 1.0 / lens.astype(jnp.float32)

    avail = jnp.maximum(end - begin, 0) // BLK
    tot_avail = jnp.maximum(jnp.sum(avail), 1)
    tc_nb = jnp.minimum(avail, (avail * NTC) // tot_avail)
    tc_nb = jnp.where(begin % BLK == 0, tc_nb, 0)
    base_blk = begin // BLK

    pos = jnp.concatenate([jnp.zeros((1,), jnp.int32),
                           jnp.cumsum(tc_nb)])[:BS]
    ii = jnp.repeat(jnp.arange(BS, dtype=jnp.int32), NTB)
    jj = jnp.tile(jnp.arange(NTB, dtype=jnp.int32), BS)
    kk = jnp.where(jj < tc_nb[ii], pos[ii] + jj, NTC)
    rows = jnp.zeros((NTC,), jnp.int32).at[kk].set(ii, mode="drop")
    blks = jnp.zeros((NTC,), jnp.int32).at[kk].set(
        base_blk[ii] + jj, mode="drop")
    ntot_s = jnp.sum(tc_nb)
    fill = jnp.maximum(ntot_s - 1, 0)
    karange = jnp.arange(NTC, dtype=jnp.int32)
    rows = jnp.where(karange < ntot_s, rows, rows[fill])
    blks = jnp.where(karange < ntot_s, blks, blks[fill])
    tc_valid = (karange < ntot_s).astype(jnp.int32)

    sc_begin = begin + tc_nb * BLK
    lens_sc = end - sc_begin
    cum = jnp.concatenate([jnp.zeros((1,), jnp.int32), jnp.cumsum(lens_sc)])
    total = cum[BS]
    pw = (jnp.arange(NSUB + 1, dtype=jnp.int32) * total) // NSUB

    def pad32(x):
        return jnp.zeros((2 * BS,), jnp.int32).at[: x.shape[0]].set(x)

    packed = jnp.stack([
        pad32(sc_begin), pad32(end),
        jnp.zeros((2 * BS,), jnp.int32),
        pad32(cum), pad32(pw),
        jnp.zeros((2 * BS,), jnp.int32),
        jnp.zeros((2 * BS,), jnp.int32),
        jnp.zeros((2 * BS,), jnp.int32),
    ])
    inv32 = jnp.zeros((2 * BS,), jnp.float32).at[:BS].set(inv_cnt)
    sc_part = _avg_sc(seq, packed, inv32)
    tc_part = _tc_blocks(seq, rows, blks, tc_valid, inv_cnt)
    return sc_part + tc_part


# speedup vs baseline: 1.1334x; 1.1334x over previous
import functools

import jax
import jax.numpy as jnp
from jax import lax
from jax.experimental import pallas as pl
from jax.experimental.pallas import tpu as pltpu
from jax.experimental.pallas import tpu_sc as plsc

BS = 16
L = 4096
D = 1024
NCORES = 2
NSUB = 16
CH = 96
DH = D // NCORES
NDB = DH // 16


def _avg_sc(seq, args, inv32):
    mesh = plsc.VectorSubcoreMesh(core_axis_name="c", subcore_axis_name="s")

    @functools.partial(
        pl.kernel,
        mesh=mesh,
        out_type=jax.ShapeDtypeStruct((BS, D), jnp.float32),
        scratch_types=[
            pltpu.VMEM((8, 2 * BS), jnp.int32),
            pltpu.VMEM((2 * BS,), jnp.float32),
            pltpu.VMEM((CH, DH), jnp.float32),
            pltpu.VMEM((CH, DH), jnp.float32),
            pltpu.VMEM((BS, DH), jnp.float32),
            pltpu.VMEM((NSUB, DH), jnp.float32),
            pltpu.VMEM_SHARED((NSUB, BS, DH), jnp.float32),
            pltpu.SemaphoreType.DMA,
            pltpu.SemaphoreType.DMA,
        ],
    )
    def k(seq_hbm, packed_hbm, inv_hbm, out_hbm,
          packed, inv_v, buf0, buf1, part, fin, shared, sem0, sem1):
        c = lax.axis_index("c")
        s = lax.axis_index("s")
        d0 = c * DH

        pltpu.sync_copy(packed_hbm, packed)
        pltpu.sync_copy(inv_hbm, inv_v)

        def ext(row, i):
            return packed[row, pl.ds(i, 16)][0]

        bg_v, en_v, cum_v, pw_v = 0, 1, 3, 4
        g0 = ext(pw_v, s)
        g1 = ext(pw_v, s + 1)

        def zero_part(r, carry):
            for db in range(NDB):
                part[r, pl.ds(db * 16, 16)] = jnp.zeros((16,), jnp.float32)
            return carry

        lax.fori_loop(0, BS, zero_part, 0)
        pltpu.sync_copy(part, shared.at[s])

        def start_dma(r, cb, buf, sem):
            pltpu.async_copy(
                seq_hbm.at[r, pl.ds(cb, CH), pl.ds(d0, DH)], buf, sem)

        def wait_dma(buf, sem):
            pltpu.make_async_copy(
                seq_hbm.at[0, pl.ds(0, CH), pl.ds(d0, DH)], buf, sem).wait()

        def chunk_base(g, base0):
            return jnp.minimum(base0 + g * CH, L - CH)

        def chunk(r, g, nch, base0, lo_abs, hi_abs, buf, sem):
            wait_dma(buf, sem)
            base = chunk_base(g, base0)
            lo = jnp.maximum(base0 + g * CH, lo_abs) - base
            hi = jnp.minimum(base0 + (g + 1) * CH, hi_abs) - base

            accs = tuple(part[r, pl.ds(db * 16, 16)] for db in range(NDB))

            def add_l(l, accs):
                return tuple(
                    a + buf[l, pl.ds(db * 16, 16)]
                    for db, a in enumerate(accs))

            n2 = (hi - lo) // 2

            def pair_body(i, accs):
                l = lo + 2 * i
                return add_l(l + 1, add_l(l, accs))

            accs = lax.fori_loop(0, n2, pair_body, accs)
            accs = lax.fori_loop(lo + 2 * n2, hi, add_l, accs)

            for db, a in enumerate(accs):
                part[r, pl.ds(db * 16, 16)] = a

            @pl.when(g + 2 < nch)
            def _():
                start_dma(r, chunk_base(g + 2, base0), buf, sem)

        def seg_bounds(r):
            S = ext(cum_v, r)
            bg_r = ext(bg_v, r)
            ln = ext(en_v, r) - bg_r
            a = jnp.maximum(g0 - S, 0)
            b = jnp.minimum(g1 - S, ln)
            return bg_r, a, b

        def seg_body(r, carry):
            bg_r, a, b = seg_bounds(r)

            @pl.when(a < b)
            def _():
                lo_abs = bg_r + a
                hi_abs = bg_r + b
                base0 = (lo_abs // 8) * 8
                nch = (hi_abs - base0 + CH - 1) // CH
                start_dma(r, chunk_base(0, base0), buf0, sem0)

                @pl.when(nch > 1)
                def _():
                    start_dma(r, chunk_base(1, base0), buf1, sem1)

                def g_body(g, carry2):
                    @pl.when(g % 2 == 0)
                    def _():
                        chunk(r, g, nch, base0, lo_abs, hi_abs, buf0, sem0)

                    @pl.when(g % 2 == 1)
                    def _():
                        chunk(r, g, nch, base0, lo_abs, hi_abs, buf1, sem1)

                    return carry2

                lax.fori_loop(0, nch, g_body, 0)

            return carry

        lax.fori_loop(0, BS, seg_body, 0)

        def copy_body(r, carry):
            _, a, b = seg_bounds(r)

            @pl.when(a < b)
            def _():
                pltpu.sync_copy(part.at[r], shared.at[s, r])

            return carry

        lax.fori_loop(0, BS, copy_body, 0)
        plsc.subcore_barrier()

        pltpu.sync_copy(shared.at[:, s, :], fin)
        inv = inv_v[pl.ds(s, 16)][0]
        for db in range(NDB):
            sl = pl.ds(db * 16, 16)
            a = fin[0, sl]
            for w in range(1, NSUB):
                a = a + fin[w, sl]
            fin[0, sl] = a * inv
        pltpu.sync_copy(fin.at[0], out_hbm.at[s, pl.ds(d0, DH)])

    return k(seq, args, inv32)


def kernel(seq, begin, end):
    begin = jnp.asarray(begin, jnp.int32)
    end = jnp.asarray(end, jnp.int32)
    lens = end - begin
    inv_cnt = 1.0 / lens.astype(jnp.float32)

    sc_begin = begin
    lens_sc = end - sc_begin
    cum = jnp.concatenate([jnp.zeros((1,), jnp.int32), jnp.cumsum(lens_sc)])
    total = cum[BS]
    pw = (jnp.arange(NSUB + 1, dtype=jnp.int32) * total) // NSUB

    def pad32(x):
        return jnp.zeros((2 * BS,), jnp.int32).at[: x.shape[0]].set(x)

    packed = jnp.stack([
        pad32(sc_begin), pad32(end),
        jnp.zeros((2 * BS,), jnp.int32),
        pad32(cum), pad32(pw),
        jnp.zeros((2 * BS,), jnp.int32),
        jnp.zeros((2 * BS,), jnp.int32),
        jnp.zeros((2 * BS,), jnp.int32),
    ])
    inv32 = jnp.zeros((2 * BS,), jnp.float32).at[:BS].set(inv_cnt)
    return _avg_sc(seq, packed, inv32)


# speedup vs baseline: 1.1511x; 1.0156x over previous
import functools

import jax
import jax.numpy as jnp
from jax import lax
from jax.experimental import pallas as pl
from jax.experimental.pallas import tpu as pltpu
from jax.experimental.pallas import tpu_sc as plsc

BS = 16
L = 4096
D = 1024
NCORES = 2
NSUB = 16
CH = 64
DH = D // NCORES
NDB = DH // 16


def _avg_sc(seq, args, inv32):
    mesh = plsc.VectorSubcoreMesh(core_axis_name="c", subcore_axis_name="s")

    @functools.partial(
        pl.kernel,
        mesh=mesh,
        out_type=jax.ShapeDtypeStruct((BS, D), jnp.float32),
        scratch_types=[
            pltpu.VMEM((8, 2 * BS), jnp.int32),
            pltpu.VMEM((2 * BS,), jnp.float32),
            pltpu.VMEM((CH, DH), jnp.float32),
            pltpu.VMEM((CH, DH), jnp.float32),
            pltpu.VMEM((BS, DH), jnp.float32),
            pltpu.VMEM((NSUB, DH), jnp.float32),
            pltpu.VMEM_SHARED((NSUB, BS, DH), jnp.float32),
            pltpu.SemaphoreType.DMA,
            pltpu.SemaphoreType.DMA,
        ],
    )
    def k(seq_hbm, packed_hbm, inv_hbm, out_hbm,
          packed, inv_v, buf0, buf1, part, fin, shared, sem0, sem1):
        c = lax.axis_index("c")
        s = lax.axis_index("s")
        d0 = c * DH

        pltpu.sync_copy(packed_hbm, packed)
        pltpu.sync_copy(inv_hbm, inv_v)

        def ext(row, i):
            return packed[row, pl.ds(i, 16)][0]

        bg_v, en_v, cum_v, pw_v = 0, 1, 3, 4
        g0 = ext(pw_v, s)
        g1 = ext(pw_v, s + 1)

        def zero_part(r, carry):
            for db in range(NDB):
                part[r, pl.ds(db * 16, 16)] = jnp.zeros((16,), jnp.float32)
            return carry

        lax.fori_loop(0, BS, zero_part, 0)
        pltpu.sync_copy(part, shared.at[s])

        def start_dma(r, cb, buf, sem):
            pltpu.async_copy(
                seq_hbm.at[r, pl.ds(cb, CH), pl.ds(d0, DH)], buf, sem)

        def wait_dma(buf, sem):
            pltpu.make_async_copy(
                seq_hbm.at[0, pl.ds(0, CH), pl.ds(d0, DH)], buf, sem).wait()

        def chunk_base(g, base0):
            return jnp.minimum(base0 + g * CH, L - CH)

        def chunk(r, g, nch, base0, lo_abs, hi_abs, buf, sem):
            wait_dma(buf, sem)
            base = chunk_base(g, base0)
            lo = jnp.maximum(base0 + g * CH, lo_abs) - base
            hi = jnp.minimum(base0 + (g + 1) * CH, hi_abs) - base

            accs = tuple(part[r, pl.ds(db * 16, 16)] for db in range(NDB))

            def add_l(l, accs):
                return tuple(
                    a + buf[l, pl.ds(db * 16, 16)]
                    for db, a in enumerate(accs))

            n2 = (hi - lo) // 2

            def pair_body(i, accs):
                l = lo + 2 * i
                return add_l(l + 1, add_l(l, accs))

            accs = lax.fori_loop(0, n2, pair_body, accs)
            accs = lax.fori_loop(lo + 2 * n2, hi, add_l, accs)

            for db, a in enumerate(accs):
                part[r, pl.ds(db * 16, 16)] = a

            @pl.when(g + 2 < nch)
            def _():
                start_dma(r, chunk_base(g + 2, base0), buf, sem)

        def seg_bounds(r):
            S = ext(cum_v, r)
            bg_r = ext(bg_v, r)
            ln = ext(en_v, r) - bg_r
            a = jnp.maximum(g0 - S, 0)
            b = jnp.minimum(g1 - S, ln)
            return bg_r, a, b

        def seg_body(r, carry):
            bg_r, a, b = seg_bounds(r)

            @pl.when(a < b)
            def _():
                lo_abs = bg_r + a
                hi_abs = bg_r + b
                base0 = (lo_abs // 8) * 8
                nch = (hi_abs - base0 + CH - 1) // CH
                start_dma(r, chunk_base(0, base0), buf0, sem0)

                @pl.when(nch > 1)
                def _():
                    start_dma(r, chunk_base(1, base0), buf1, sem1)

                def g_body(g, carry2):
                    @pl.when(g % 2 == 0)
                    def _():
                        chunk(r, g, nch, base0, lo_abs, hi_abs, buf0, sem0)

                    @pl.when(g % 2 == 1)
                    def _():
                        chunk(r, g, nch, base0, lo_abs, hi_abs, buf1, sem1)

                    return carry2

                lax.fori_loop(0, nch, g_body, 0)

            return carry

        lax.fori_loop(0, BS, seg_body, 0)

        def copy_body(r, carry):
            _, a, b = seg_bounds(r)

            @pl.when(a < b)
            def _():
                pltpu.sync_copy(part.at[r], shared.at[s, r])

            return carry

        lax.fori_loop(0, BS, copy_body, 0)
        plsc.subcore_barrier()

        pltpu.sync_copy(shared.at[:, s, :], fin)
        inv = inv_v[pl.ds(s, 16)][0]
        for db in range(NDB):
            sl = pl.ds(db * 16, 16)
            a = fin[0, sl]
            for w in range(1, NSUB):
                a = a + fin[w, sl]
            fin[0, sl] = a * inv
        pltpu.sync_copy(fin.at[0], out_hbm.at[s, pl.ds(d0, DH)])

    return k(seq, args, inv32)


def kernel(seq, begin, end):
    begin = jnp.asarray(begin, jnp.int32)
    end = jnp.asarray(end, jnp.int32)
    lens = end - begin
    inv_cnt = 1.0 / lens.astype(jnp.float32)

    sc_begin = begin
    lens_sc = end - sc_begin
    cum = jnp.concatenate([jnp.zeros((1,), jnp.int32), jnp.cumsum(lens_sc)])
    total = cum[BS]
    pw = (jnp.arange(NSUB + 1, dtype=jnp.int32) * total) // NSUB

    def pad32(x):
        return jnp.zeros((2 * BS,), jnp.int32).at[: x.shape[0]].set(x)

    packed = jnp.stack([
        pad32(sc_begin), pad32(end),
        jnp.zeros((2 * BS,), jnp.int32),
        pad32(cum), pad32(pw),
        jnp.zeros((2 * BS,), jnp.int32),
        jnp.zeros((2 * BS,), jnp.int32),
        jnp.zeros((2 * BS,), jnp.int32),
    ])
    inv32 = jnp.zeros((2 * BS,), jnp.float32).at[:BS].set(inv_cnt)
    return _avg_sc(seq, packed, inv32)
